# Initial kernel scaffold; baseline (speedup 1.0000x reference)
import jax
import jax.numpy as jnp
from jax.experimental import pallas as pl


def kernel(input_features, edge_features, edge_index, W_emb, b_emb, W1, al1, ar1, W2, al2, ar2, Ws, bs):
    raise NotImplementedError("write your pallas kernel here")


# speedup vs baseline: 13.1966x; 13.1966x over previous
import functools

import jax
import jax.numpy as jnp
from jax import lax
from jax.experimental import pallas as pl
from jax.experimental.pallas import tpu as pltpu
from jax.experimental.pallas import tpu_sc as plsc

N_NODES = 10000
N_EDGES = 320000
GW = 144
NC, NS, L = 2, 16, 16
NW = NC * NS
B = 80
EPW = N_EDGES // NW
NCHUNK = EPW // B
SROW = 128
RPT = 5 * SROW
NPAD = NS * RPT
BN = 1000

_mesh = plsc.VectorSubcoreMesh(core_axis_name="c", subcore_axis_name="s")


def _edge_pass_body(n_heads, g_hbm, r_hbm, ei_hbm, zeros_hbm, acc_hbm,
                    src_v, dst_v, rows_v, er_v, stage_v, accum, sem_g, sem_r):
    cid = lax.axis_index("c")
    sid = lax.axis_index("s")
    wid = sid * NC + cid
    dim = 128 // n_heads
    row0 = pl.multiple_of(sid * RPT, 8)
    pltpu.sync_copy(zeros_hbm, stage_v)
    for k in range(RPT // SROW):
        pltpu.sync_copy(stage_v, accum.at[pl.ds(row0 + k * SROW, SROW)])
    plsc.subcore_barrier()

    def chunk(i, carry):
        base = pl.multiple_of(wid * EPW + i * B, 8)
        pltpu.sync_copy(ei_hbm.at[0, pl.ds(base, B)], src_v)
        pltpu.sync_copy(ei_hbm.at[1, pl.ds(base, B)], dst_v)
        cp_g = pltpu.async_copy(g_hbm.at[src_v], rows_v, sem_g)
        cp_r = pltpu.async_copy(r_hbm.at[dst_v], er_v, sem_r)
        cp_g.wait()
        cp_r.wait()
        for j in range(B // L):
            row_ids = j * L + lax.iota(jnp.int32, L)
            ws = []
            for hh in range(n_heads):
                col_el = jnp.full((L,), 128 + hh, jnp.int32)
                elh = plsc.load_gather(rows_v, (row_ids, col_el))
                erh = plsc.load_gather(er_v, (row_ids, jnp.full((L,), hh, jnp.int32)))
                e = elh + erh
                e = jnp.maximum(e, 0.2 * e)
                w = jnp.exp(e)
                ws.append(w)
                plsc.store_scatter(rows_v, (row_ids, col_el), w)
            for c in range(128):
                colv = jnp.full((L,), c, jnp.int32)
                vals = plsc.load_gather(rows_v, (row_ids, colv))
                plsc.store_scatter(rows_v, (row_ids, colv), vals * ws[c // dim])
        pltpu.sync_copy(rows_v, accum.at[dst_v], add=True)
        return carry

    lax.fori_loop(0, NCHUNK, chunk, 0)
    plsc.subcore_barrier()
    for k in range(RPT // SROW):
        pltpu.sync_copy(accum.at[pl.ds(row0 + k * SROW, SROW)], stage_v)
        pltpu.sync_copy(stage_v, acc_hbm.at[cid, pl.ds(row0 + k * SROW, SROW)])


def _make_edge_pass(n_heads):
    return functools.partial(
        pl.kernel,
        out_type=jax.ShapeDtypeStruct((NC, NPAD, GW), jnp.float32),
        mesh=_mesh,
        compiler_params=pltpu.CompilerParams(use_tc_tiling_on_sc=False, needs_layout_passes=False),
        scratch_types=[
            pltpu.VMEM((B,), jnp.int32),
            pltpu.VMEM((B,), jnp.int32),
            pltpu.VMEM((B, GW), jnp.float32),
            pltpu.VMEM((B, L), jnp.float32),
            pltpu.VMEM((SROW, GW), jnp.float32),
            pltpu.VMEM_SHARED((NPAD, GW), jnp.float32),
            pltpu.SemaphoreType.DMA,
            pltpu.SemaphoreType.DMA,
        ],
    )(functools.partial(_edge_pass_body, n_heads))


_edge_pass_h2 = _make_edge_pass(2)
_edge_pass_h1 = _make_edge_pass(1)


@functools.partial(
    pl.kernel,
    out_type=jax.ShapeDtypeStruct((N_EDGES, L), jnp.float32),
    mesh=_mesh,
    compiler_params=pltpu.CompilerParams(use_tc_tiling_on_sc=False, needs_layout_passes=False),
    scratch_types=[
        pltpu.VMEM((B,), jnp.int32),
        pltpu.VMEM((B,), jnp.int32),
        pltpu.VMEM((B, L), jnp.float32),
        pltpu.VMEM((B, L), jnp.float32),
        pltpu.VMEM((B, L), jnp.float32),
        pltpu.SemaphoreType.DMA,
        pltpu.SemaphoreType.DMA,
    ],
)
def _edge_score(p_hbm, q_hbm, ei_hbm, out_hbm,
                src_v, dst_v, p_v, q_v, o_v, sem_p, sem_q):
    cid = lax.axis_index("c")
    sid = lax.axis_index("s")
    wid = sid * NC + cid

    def chunk(i, carry):
        base = pl.multiple_of(wid * EPW + i * B, 8)
        pltpu.sync_copy(ei_hbm.at[0, pl.ds(base, B)], src_v)
        pltpu.sync_copy(ei_hbm.at[1, pl.ds(base, B)], dst_v)
        cp_p = pltpu.async_copy(p_hbm.at[src_v], p_v, sem_p)
        cp_q = pltpu.async_copy(q_hbm.at[dst_v], q_v, sem_q)
        cp_p.wait()
        cp_q.wait()
        for j in range(B):
            o_v[j] = p_v[j] + q_v[j]
        pltpu.sync_copy(o_v, out_hbm.at[pl.ds(pl.multiple_of(wid * EPW + i * B, 8), B)])
        return carry

    lax.fori_loop(0, NCHUNK, chunk, 0)


def _tc1_body(x_ref, wemb_ref, bemb_ref, w1_ref, al1_ref, ar1_ref, g_ref, r_ref):
    h = jnp.dot(x_ref[...], wemb_ref[...], preferred_element_type=jnp.float32)
    h = h + bemb_ref[...][None, :]
    feat = jnp.dot(h, w1_ref[...], preferred_element_type=jnp.float32)
    al = al1_ref[...]
    ar = ar1_ref[...]
    el0 = jnp.sum(feat[:, :64] * al[0][None, :], axis=1, keepdims=True)
    el1 = jnp.sum(feat[:, 64:] * al[1][None, :], axis=1, keepdims=True)
    er0 = jnp.sum(feat[:, :64] * ar[0][None, :], axis=1, keepdims=True)
    er1 = jnp.sum(feat[:, 64:] * ar[1][None, :], axis=1, keepdims=True)
    z14 = jnp.zeros((BN, 14), jnp.float32)
    g_ref[...] = jnp.concatenate([feat, el0, el1, z14], axis=1)
    r_ref[...] = jnp.concatenate([er0, er1, z14], axis=1)


def _tc2_body(acc_ref, w2_ref, al2_ref, ar2_ref, g_ref, r_ref):
    s = acc_ref[0] + acc_ref[1]
    num = s[:, :128]
    den0 = s[:, 128:129] + 1e-9
    den1 = s[:, 129:130] + 1e-9
    rst = jnp.concatenate([num[:, :64] / den0, num[:, 64:] / den1], axis=1)
    h = jnp.where(rst > 0, rst, jnp.exp(rst) - 1.0)
    feat = jnp.dot(h, w2_ref[...], preferred_element_type=jnp.float32)
    el = jnp.sum(feat * al2_ref[...][0][None, :], axis=1, keepdims=True)
    er = jnp.sum(feat * ar2_ref[...][0][None, :], axis=1, keepdims=True)
    z15 = jnp.zeros((BN, 15), jnp.float32)
    g_ref[...] = jnp.concatenate([feat, el, z15], axis=1)
    r_ref[...] = jnp.concatenate([er, z15], axis=1)


def _tc3_body(acc_ref, ws_ref, bs_ref, p_ref, q_ref):
    s = acc_ref[0] + acc_ref[1]
    num = s[:, :128]
    den = s[:, 128:129] + 1e-9
    rst = num / den
    h = jnp.where(rst > 0, rst, jnp.exp(rst) - 1.0)
    p = jnp.dot(h, ws_ref[:128, :], preferred_element_type=jnp.float32)
    p = p + bs_ref[...][None, :]
    q = jnp.dot(h, ws_ref[128:, :], preferred_element_type=jnp.float32)
    z6 = jnp.zeros((BN, 6), jnp.float32)
    p_ref[...] = jnp.concatenate([p, z6], axis=1)
    q_ref[...] = jnp.concatenate([q, z6], axis=1)


def _full(i):
    return (0, 0)


def _tc1(x, wemb, bemb, w1, al1, ar1):
    return pl.pallas_call(
        _tc1_body,
        grid=(N_NODES // BN,),
        in_specs=[
            pl.BlockSpec((BN, 128), lambda i: (i, 0)),
            pl.BlockSpec((128, 128), _full),
            pl.BlockSpec((128,), lambda i: (0,)),
            pl.BlockSpec((128, 128), _full),
            pl.BlockSpec((2, 64), _full),
            pl.BlockSpec((2, 64), _full),
        ],
        out_specs=[
            pl.BlockSpec((BN, GW), lambda i: (i, 0)),
            pl.BlockSpec((BN, L), lambda i: (i, 0)),
        ],
        out_shape=[
            jax.ShapeDtypeStruct((N_NODES, GW), jnp.float32),
            jax.ShapeDtypeStruct((N_NODES, L), jnp.float32),
        ],
    )(x, wemb, bemb, w1, al1, ar1)


def _tc2(acc, w2, al2, ar2):
    return pl.pallas_call(
        _tc2_body,
        grid=(N_NODES // BN,),
        in_specs=[
            pl.BlockSpec((NC, BN, GW), lambda i: (0, i, 0)),
            pl.BlockSpec((128, 128), _full),
            pl.BlockSpec((1, 128), _full),
            pl.BlockSpec((1, 128), _full),
        ],
        out_specs=[
            pl.BlockSpec((BN, GW), lambda i: (i, 0)),
            pl.BlockSpec((BN, L), lambda i: (i, 0)),
        ],
        out_shape=[
            jax.ShapeDtypeStruct((N_NODES, GW), jnp.float32),
            jax.ShapeDtypeStruct((N_NODES, L), jnp.float32),
        ],
    )(acc, w2, al2, ar2)


def _tc3(acc, ws, bs):
    return pl.pallas_call(
        _tc3_body,
        grid=(N_NODES // BN,),
        in_specs=[
            pl.BlockSpec((NC, BN, GW), lambda i: (0, i, 0)),
            pl.BlockSpec((256, 10), _full),
            pl.BlockSpec((10,), lambda i: (0,)),
        ],
        out_specs=[
            pl.BlockSpec((BN, L), lambda i: (i, 0)),
            pl.BlockSpec((BN, L), lambda i: (i, 0)),
        ],
        out_shape=[
            jax.ShapeDtypeStruct((N_NODES, L), jnp.float32),
            jax.ShapeDtypeStruct((N_NODES, L), jnp.float32),
        ],
    )(acc, ws, bs)


def kernel(input_features, edge_features, edge_index, W_emb, b_emb,
           W1, al1, ar1, W2, al2, ar2, Ws, bs):
    del edge_features
    ei = edge_index.astype(jnp.int32)
    zeros_acc = jnp.zeros((SROW, GW), jnp.float32)

    g1, r1 = _tc1(input_features, W_emb, b_emb, W1, al1, ar1)
    acc1 = _edge_pass_h2(g1, r1, ei, zeros_acc)
    g2, r2 = _tc2(acc1, W2, al2, ar2)
    acc2 = _edge_pass_h1(g2, r2, ei, zeros_acc)
    p, q = _tc3(acc2, Ws, bs)
    out16 = _edge_score(p, q, ei)
    return out16[:, :10]
